# baseline (speedup 1.0000x reference)
import functools

import jax
import jax.numpy as jnp
from jax import lax
from jax.experimental import pallas as pl
from jax.experimental.pallas import tpu as pltpu
from jax.experimental.pallas import tpu_sc as plsc

NC = 2
NS = 16
NW = NC * NS
KCH = 80


def _tile_rows_copy(N, s, copy_fn):
    RA = (N // (8 * NS)) * 8
    tail = N - RA * NS
    copy_fn(pl.ds(s * RA, RA))
    if tail:
        @pl.when(s == 0)
        def _():
            copy_fn(pl.ds(NS * RA, tail))

def _make_sc_scatter(N, EP, H):
    EW = EP // NW
    J = EW // KCH
    JH = J // 2
    mesh = plsc.VectorSubcoreMesh(core_axis_name="c", subcore_axis_name="s")

    @functools.partial(
        pl.kernel,
        out_type=jax.ShapeDtypeStruct((NC, N, H), jnp.float32),
        mesh=mesh,
        scratch_types=[
            pltpu.VMEM((EW,), jnp.int32),
            pltpu.VMEM((J, KCH), jnp.int32),
            pltpu.VMEM((KCH, H), jnp.float32),
            pltpu.VMEM((KCH, H), jnp.float32),
            pltpu.VMEM_SHARED((N, H), jnp.float32),
            pltpu.SemaphoreType.DMA,
            pltpu.SemaphoreType.DMA,
        ],
    )
    def scatter_kernel(xs_hbm, zeros_hbm, srcf_hbm, dst_hbm, out_hbm,
                       src_v, dst_v, rows0, rows1, acc, sem0, sem1):
        c = lax.axis_index("c")
        s = lax.axis_index("s")
        w = c * NS + s

        @pl.when(c == 0)
        def _():
            _tile_rows_copy(N, s, lambda r: pltpu.sync_copy(
                xs_hbm.at[r], acc.at[r]))

        @pl.when(c != 0)
        def _():
            _tile_rows_copy(N, s, lambda r: pltpu.sync_copy(
                zeros_hbm.at[r], acc.at[r]))

        plsc.subcore_barrier()

        pltpu.sync_copy(srcf_hbm.at[w], src_v)
        pltpu.sync_copy(dst_hbm.at[w], dst_v)

        def gather(j, buf, sem):
            pltpu.async_copy(
                xs_hbm.at[src_v.at[pl.ds(j * KCH, KCH)]], buf, sem)

        def drain_scatter(j, buf, sem):
            pltpu.make_async_copy(
                xs_hbm.at[src_v.at[pl.ds(j * KCH, KCH)]], buf, sem).wait()
            pltpu.sync_copy(buf, acc.at[dst_v.at[j]], add=True)

        gather(0, rows0, sem0)

        def body(jg, carry):
            gather(2 * jg + 1, rows1, sem1)
            drain_scatter(2 * jg, rows0, sem0)
            gather(2 * jg + 2, rows0, sem0)
            drain_scatter(2 * jg + 1, rows1, sem1)
            return carry

        lax.fori_loop(0, (J - 1) // 2, body, 0)
        drain_scatter(J - 1, rows0, sem0)
        plsc.subcore_barrier()
        _tile_rows_copy(N, s, lambda r: pltpu.sync_copy(
            acc.at[r], out_hbm.at[c, r]))

    return scatter_kernel


def _make_sc_degree(N, E):
    EW = E // NW
    J = EW // KCH
    mesh = plsc.VectorSubcoreMesh(core_axis_name="c", subcore_axis_name="s")

    @functools.partial(
        pl.kernel,
        out_type=jax.ShapeDtypeStruct((NC, N, 8), jnp.float32),
        mesh=mesh,
        scratch_types=[
            pltpu.VMEM((J, KCH), jnp.int32),
            pltpu.VMEM((KCH, 8), jnp.float32),
            pltpu.VMEM_SHARED((N, 8), jnp.float32),
            pltpu.SemaphoreType.DMA,
        ],
    )
    def degree_kernel(init_hbm, onesk_hbm, dst_hbm, out_hbm,
                      dst_v, ones_v, acc, sem):
        c = lax.axis_index("c")
        s = lax.axis_index("s")
        w = c * NS + s

        _tile_rows_copy(N, s, lambda r: pltpu.sync_copy(
            init_hbm.at[c].at[r], acc.at[r]))

        pltpu.sync_copy(onesk_hbm, ones_v)
        pltpu.sync_copy(dst_hbm.at[w], dst_v)
        plsc.subcore_barrier()

        def body(j, carry):
            pltpu.async_copy(ones_v, acc.at[dst_v.at[j]], sem, add=True)
            return carry

        lax.fori_loop(0, J, body, 0)

        def drain(j, carry):
            pltpu.make_async_copy(ones_v, acc.at[dst_v.at[j]], sem).wait()
            return carry

        lax.fori_loop(0, J, drain, 0)
        plsc.subcore_barrier()
        _tile_rows_copy(N, s, lambda r: pltpu.sync_copy(
            acc.at[r], out_hbm.at[c, r]))

    return degree_kernel


_BR = 1000


def _row(H):
    return pl.BlockSpec((_BR, H), lambda i: (i, 0))


def _full(shape):
    return pl.BlockSpec(shape, lambda i: (0,) * len(shape))


def _acc2(H):
    return pl.BlockSpec((NC, _BR, H), lambda i: (0, i, 0))


def _tc_in_body(x_ref, win_ref, bin_ref, wc0_ref, dega_ref, degb_ref,
                h_ref, dinv_ref, xs_ref):
    h = jnp.dot(x_ref[...], win_ref[...],
                preferred_element_type=jnp.float32) + bin_ref[...]
    dinv = lax.rsqrt(dega_ref[...] + degb_ref[...])
    h_ref[...] = h
    dinv_ref[...] = dinv
    xs_ref[...] = dinv * jnp.dot(h, wc0_ref[...],
                                 preferred_element_type=jnp.float32)


def _tc_mid_body(has_xl, acc_ref, dinv_ref, bc_ref, h_ref, wl_ref, bl_ref,
                 *rest):
    if has_xl:
        xl_ref, wc_ref, hn_ref, xln_ref, xs_ref = rest
    else:
        wc_ref, hn_ref, xln_ref, xs_ref = rest
    dinv = dinv_ref[...]
    conv = dinv * (acc_ref[0] + acc_ref[1]) + bc_ref[...]
    lin = jnp.dot(h_ref[...], wl_ref[...],
                  preferred_element_type=jnp.float32) + bl_ref[...]
    hn = jnp.maximum(conv + lin, 0.0)
    xln = (xl_ref[...] + hn) if has_xl else hn
    hn_ref[...] = hn
    xln_ref[...] = xln
    xs_ref[...] = dinv * jnp.dot(hn, wc_ref[...],
                                 preferred_element_type=jnp.float32)


def _tc_out_body(acc_ref, dinv_ref, bc_ref, h_ref, wl_ref, bl_ref, xl_ref,
                 wp_ref, bp_ref, out_ref):
    conv = dinv_ref[...] * (acc_ref[0] + acc_ref[1]) + bc_ref[...]
    lin = jnp.dot(h_ref[...], wl_ref[...],
                  preferred_element_type=jnp.float32) + bl_ref[...]
    hn = jnp.maximum(conv + lin, 0.0)
    xl = xl_ref[...] + hn
    out_ref[...] = jnp.dot(xl, wp_ref[...],
                           preferred_element_type=jnp.float32) + bp_ref[...]


def kernel(x, edge_index, W_in, b_in, Wc, bc, Wl, bl, Wp, bp):
    N, _ = x.shape
    H = W_in.shape[1]
    E = edge_index.shape[1]
    DP = Wp.shape[1]
    grid = (N // _BR,)

    J = E // (NW * KCH)
    EP = J * NW * KCH
    src_i = edge_index[0].astype(jnp.int32)
    dst_i = edge_index[1].astype(jnp.int32)
    src = src_i.reshape(NW, E // NW)
    dst = dst_i.reshape(NW, J, KCH)
    zeros_nh = jnp.zeros((N, H), jnp.float32)
    init8 = jnp.stack([jnp.ones((N, 8), jnp.float32),
                       jnp.zeros((N, 8), jnp.float32)])
    ones_k8 = jnp.ones((KCH, 8), jnp.float32)

    sc_degree = _make_sc_degree(N, E)
    sc_scatter = _make_sc_scatter(N, EP, H)

    f32 = jnp.float32
    nh = jax.ShapeDtypeStruct((N, H), f32)

    deg2 = sc_degree(init8, ones_k8, dst)
    dega = deg2[0, :, 0:1]
    degb = deg2[1, :, 0:1]

    h0, dinv, xs = pl.pallas_call(
        _tc_in_body,
        grid=grid,
        in_specs=[_row(H), _full((H, H)), _full((1, H)), _full((H, H)),
                  _row(1), _row(1)],
        out_specs=[_row(H), _row(1), _row(H)],
        out_shape=[nh, jax.ShapeDtypeStruct((N, 1), f32), nh],
    )(x, W_in, b_in.reshape(1, H), Wc[0], dega, degb)

    h, xl = h0, None
    for i in range(2):
        acc = sc_scatter(xs, zeros_nh, src, dst)
        ins = [acc, dinv, bc[i].reshape(1, H), h, Wl[i], bl[i].reshape(1, H)]
        specs = [_acc2(H), _row(1), _full((1, H)), _row(H), _full((H, H)),
                 _full((1, H))]
        if xl is not None:
            ins.append(xl)
            specs.append(_row(H))
        ins.append(Wc[i + 1])
        specs.append(_full((H, H)))
        h, xl, xs = pl.pallas_call(
            functools.partial(_tc_mid_body, xl is not None),
            grid=grid,
            in_specs=specs,
            out_specs=[_row(H), _row(H), _row(H)],
            out_shape=[nh, nh, nh],
        )(*ins)

    acc = sc_scatter(xs, zeros_nh, src, dst)
    out = pl.pallas_call(
        _tc_out_body,
        grid=grid,
        in_specs=[_acc2(H), _row(1), _full((1, H)), _row(H), _full((H, H)),
                  _full((1, H)), _row(H), _full((H, DP)), _full((1, DP))],
        out_specs=_row(DP),
        out_shape=jax.ShapeDtypeStruct((N, DP), f32),
    )(acc, dinv, bc[2].reshape(1, H), h, Wl[2], bl[2].reshape(1, H), xl,
      Wp, bp.reshape(1, DP))
    return out


# speedup vs baseline: 1.0180x; 1.0180x over previous
import functools

import jax
import jax.numpy as jnp
from jax import lax
from jax.experimental import pallas as pl
from jax.experimental.pallas import tpu as pltpu
from jax.experimental.pallas import tpu_sc as plsc

NC = 2
NS = 16
NW = NC * NS
KCH = 80


def _tile_rows_copy(N, s, copy_fn):
    RA = (N // (8 * NS)) * 8
    tail = N - RA * NS
    copy_fn(pl.ds(s * RA, RA))
    if tail:
        @pl.when(s == 0)
        def _():
            copy_fn(pl.ds(NS * RA, tail))

def _make_sc_scatter(N, EP, H):
    EW = EP // NW
    J = EW // KCH
    mesh = plsc.VectorSubcoreMesh(core_axis_name="c", subcore_axis_name="s")

    @functools.partial(
        pl.kernel,
        out_type=jax.ShapeDtypeStruct((NC, N, H), jnp.float32),
        mesh=mesh,
        scratch_types=[
            pltpu.VMEM((EW,), jnp.int32),
            pltpu.VMEM((J, KCH), jnp.int32),
            pltpu.VMEM((KCH, H), jnp.float32),
            pltpu.VMEM((KCH, H), jnp.float32),
            pltpu.VMEM_SHARED((N, H), jnp.float32),
            pltpu.SemaphoreType.DMA,
            pltpu.SemaphoreType.DMA,
        ],
    )
    def scatter_kernel(xs_hbm, zeros_hbm, srcf_hbm, dst_hbm, out_hbm,
                       src_v, dst_v, rows0, rows1, acc, sem0, sem1):
        c = lax.axis_index("c")
        s = lax.axis_index("s")
        w = c * NS + s

        @pl.when(c == 0)
        def _():
            _tile_rows_copy(N, s, lambda r: pltpu.sync_copy(
                xs_hbm.at[r], acc.at[r]))

        @pl.when(c != 0)
        def _():
            _tile_rows_copy(N, s, lambda r: pltpu.sync_copy(
                zeros_hbm.at[r], acc.at[r]))

        plsc.subcore_barrier()

        pltpu.sync_copy(srcf_hbm.at[w], src_v)
        pltpu.sync_copy(dst_hbm.at[w], dst_v)

        def gather(j, buf, sem):
            pltpu.async_copy(
                xs_hbm.at[src_v.at[pl.ds(j * KCH, KCH)]], buf, sem)

        def drain_scatter(j, buf, sem):
            pltpu.make_async_copy(
                xs_hbm.at[src_v.at[pl.ds(j * KCH, KCH)]], buf, sem).wait()
            pltpu.sync_copy(buf, acc.at[dst_v.at[j]], add=True)

        gather(0, rows0, sem0)

        def body(jg, carry):
            gather(2 * jg + 1, rows1, sem1)
            drain_scatter(2 * jg, rows0, sem0)
            gather(2 * jg + 2, rows0, sem0)
            drain_scatter(2 * jg + 1, rows1, sem1)
            return carry

        lax.fori_loop(0, (J - 1) // 2, body, 0)
        drain_scatter(J - 1, rows0, sem0)
        plsc.subcore_barrier()
        _tile_rows_copy(N, s, lambda r: pltpu.sync_copy(
            acc.at[r], out_hbm.at[c, r]))

    return scatter_kernel


def _make_sc_degree(N, E):
    EW = E // NW
    J = EW // KCH
    mesh = plsc.VectorSubcoreMesh(core_axis_name="c", subcore_axis_name="s")

    @functools.partial(
        pl.kernel,
        out_type=jax.ShapeDtypeStruct((NC, N, 8), jnp.float32),
        mesh=mesh,
        scratch_types=[
            pltpu.VMEM((J, KCH), jnp.int32),
            pltpu.VMEM((KCH, 8), jnp.float32),
            pltpu.VMEM_SHARED((N, 8), jnp.float32),
            pltpu.SemaphoreType.DMA,
        ],
    )
    def degree_kernel(init_hbm, onesk_hbm, dst_hbm, out_hbm,
                      dst_v, ones_v, acc, sem):
        c = lax.axis_index("c")
        s = lax.axis_index("s")
        w = c * NS + s

        _tile_rows_copy(N, s, lambda r: pltpu.sync_copy(
            init_hbm.at[c].at[r], acc.at[r]))

        pltpu.sync_copy(onesk_hbm, ones_v)
        pltpu.sync_copy(dst_hbm.at[w], dst_v)
        plsc.subcore_barrier()

        def body(j, carry):
            pltpu.async_copy(ones_v, acc.at[dst_v.at[j]], sem, add=True)
            return carry

        lax.fori_loop(0, J, body, 0)

        def drain(j, carry):
            pltpu.make_async_copy(ones_v, acc.at[dst_v.at[j]], sem).wait()
            return carry

        lax.fori_loop(0, J, drain, 0)
        plsc.subcore_barrier()
        _tile_rows_copy(N, s, lambda r: pltpu.sync_copy(
            acc.at[r], out_hbm.at[c, r]))

    return degree_kernel


_BR = 2000


def _row(H):
    return pl.BlockSpec((_BR, H), lambda i: (i, 0))


def _full(shape):
    return pl.BlockSpec(shape, lambda i: (0,) * len(shape))


def _acc2(H):
    return pl.BlockSpec((NC, _BR, H), lambda i: (0, i, 0))


def _tc_in_body(x_ref, win_ref, bin_ref, wc0_ref, dega_ref, degb_ref,
                h_ref, dinv_ref, xs_ref):
    h = jnp.dot(x_ref[...], win_ref[...],
                preferred_element_type=jnp.float32) + bin_ref[...]
    dinv = lax.rsqrt(dega_ref[...] + degb_ref[...])
    h_ref[...] = h
    dinv_ref[...] = dinv
    xs_ref[...] = dinv * jnp.dot(h, wc0_ref[...],
                                 preferred_element_type=jnp.float32)


def _tc_mid_body(has_xl, acc_ref, dinv_ref, bc_ref, h_ref, wl_ref, bl_ref,
                 *rest):
    if has_xl:
        xl_ref, wc_ref, hn_ref, xln_ref, xs_ref = rest
    else:
        wc_ref, hn_ref, xln_ref, xs_ref = rest
    dinv = dinv_ref[...]
    conv = dinv * (acc_ref[0] + acc_ref[1]) + bc_ref[...]
    lin = jnp.dot(h_ref[...], wl_ref[...],
                  preferred_element_type=jnp.float32) + bl_ref[...]
    hn = jnp.maximum(conv + lin, 0.0)
    xln = (xl_ref[...] + hn) if has_xl else hn
    hn_ref[...] = hn
    xln_ref[...] = xln
    xs_ref[...] = dinv * jnp.dot(hn, wc_ref[...],
                                 preferred_element_type=jnp.float32)


def _tc_out_body(acc_ref, dinv_ref, bc_ref, h_ref, wl_ref, bl_ref, xl_ref,
                 wp_ref, bp_ref, out_ref):
    conv = dinv_ref[...] * (acc_ref[0] + acc_ref[1]) + bc_ref[...]
    lin = jnp.dot(h_ref[...], wl_ref[...],
                  preferred_element_type=jnp.float32) + bl_ref[...]
    hn = jnp.maximum(conv + lin, 0.0)
    xl = xl_ref[...] + hn
    out_ref[...] = jnp.dot(xl, wp_ref[...],
                           preferred_element_type=jnp.float32) + bp_ref[...]


def kernel(x, edge_index, W_in, b_in, Wc, bc, Wl, bl, Wp, bp):
    N, _ = x.shape
    H = W_in.shape[1]
    E = edge_index.shape[1]
    DP = Wp.shape[1]
    grid = (N // _BR,)

    J = E // (NW * KCH)
    EP = J * NW * KCH
    src_i = edge_index[0].astype(jnp.int32)
    dst_i = edge_index[1].astype(jnp.int32)
    src = src_i.reshape(NW, E // NW)
    dst = dst_i.reshape(NW, J, KCH)
    zeros_nh = jnp.zeros((N, H), jnp.float32)
    init8 = jnp.stack([jnp.ones((N, 8), jnp.float32),
                       jnp.zeros((N, 8), jnp.float32)])
    ones_k8 = jnp.ones((KCH, 8), jnp.float32)

    sc_degree = _make_sc_degree(N, E)
    sc_scatter = _make_sc_scatter(N, EP, H)

    f32 = jnp.float32
    nh = jax.ShapeDtypeStruct((N, H), f32)

    deg2 = sc_degree(init8, ones_k8, dst)
    dega = deg2[0, :, 0:1]
    degb = deg2[1, :, 0:1]

    h0, dinv, xs = pl.pallas_call(
        _tc_in_body,
        grid=grid,
        in_specs=[_row(H), _full((H, H)), _full((1, H)), _full((H, H)),
                  _row(1), _row(1)],
        out_specs=[_row(H), _row(1), _row(H)],
        out_shape=[nh, jax.ShapeDtypeStruct((N, 1), f32), nh],
    )(x, W_in, b_in.reshape(1, H), Wc[0], dega, degb)

    h, xl = h0, None
    for i in range(2):
        acc = sc_scatter(xs, zeros_nh, src, dst)
        ins = [acc, dinv, bc[i].reshape(1, H), h, Wl[i], bl[i].reshape(1, H)]
        specs = [_acc2(H), _row(1), _full((1, H)), _row(H), _full((H, H)),
                 _full((1, H))]
        if xl is not None:
            ins.append(xl)
            specs.append(_row(H))
        ins.append(Wc[i + 1])
        specs.append(_full((H, H)))
        h, xl, xs = pl.pallas_call(
            functools.partial(_tc_mid_body, xl is not None),
            grid=grid,
            in_specs=specs,
            out_specs=[_row(H), _row(H), _row(H)],
            out_shape=[nh, nh, nh],
        )(*ins)

    acc = sc_scatter(xs, zeros_nh, src, dst)
    out = pl.pallas_call(
        _tc_out_body,
        grid=grid,
        in_specs=[_acc2(H), _row(1), _full((1, H)), _row(H), _full((H, H)),
                  _full((1, H)), _row(H), _full((H, DP)), _full((1, DP))],
        out_specs=_row(DP),
        out_shape=jax.ShapeDtypeStruct((N, DP), f32),
    )(acc, dinv, bc[2].reshape(1, H), h, Wl[2], bl[2].reshape(1, H), xl,
      Wp, bp.reshape(1, DP))
    return out


# speedup vs baseline: 1.0320x; 1.0137x over previous
import functools

import jax
import jax.numpy as jnp
from jax import lax
from jax.experimental import pallas as pl
from jax.experimental.pallas import tpu as pltpu
from jax.experimental.pallas import tpu_sc as plsc

NC = 2
NS = 16
NW = NC * NS
KCH = 80


def _tile_rows_copy(N, s, copy_fn):
    RA = (N // (8 * NS)) * 8
    tail = N - RA * NS
    copy_fn(pl.ds(s * RA, RA))
    if tail:
        @pl.when(s == 0)
        def _():
            copy_fn(pl.ds(NS * RA, tail))

def _make_sc_scatter(N, EP, H):
    EW = EP // NW
    J = EW // KCH
    mesh = plsc.VectorSubcoreMesh(core_axis_name="c", subcore_axis_name="s")

    @functools.partial(
        pl.kernel,
        out_type=jax.ShapeDtypeStruct((NC, N, H), jnp.float32),
        mesh=mesh,
        scratch_types=[
            pltpu.VMEM((EW,), jnp.int32),
            pltpu.VMEM((J, KCH), jnp.int32),
            pltpu.VMEM((KCH, H), jnp.float32),
            pltpu.VMEM((KCH, H), jnp.float32),
            pltpu.VMEM_SHARED((N, H), jnp.float32),
            pltpu.SemaphoreType.DMA,
            pltpu.SemaphoreType.DMA,
        ],
    )
    def scatter_kernel(xs_hbm, zeros_hbm, srcf_hbm, dst_hbm, out_hbm,
                       src_v, dst_v, rows0, rows1, acc, sem0, sem1):
        c = lax.axis_index("c")
        s = lax.axis_index("s")
        w = c * NS + s

        pltpu.sync_copy(srcf_hbm.at[w], src_v)
        pltpu.sync_copy(dst_hbm.at[w], dst_v)

        def gather(j, buf, sem):
            pltpu.async_copy(
                xs_hbm.at[src_v.at[pl.ds(j * KCH, KCH)]], buf, sem)

        def drain_scatter(j, buf, sem):
            pltpu.make_async_copy(
                xs_hbm.at[src_v.at[pl.ds(j * KCH, KCH)]], buf, sem).wait()
            pltpu.sync_copy(buf, acc.at[dst_v.at[j]], add=True)

        gather(0, rows0, sem0)
        gather(1, rows1, sem1)

        @pl.when(c == 0)
        def _():
            _tile_rows_copy(N, s, lambda r: pltpu.sync_copy(
                xs_hbm.at[r], acc.at[r]))

        @pl.when(c != 0)
        def _():
            _tile_rows_copy(N, s, lambda r: pltpu.sync_copy(
                zeros_hbm.at[r], acc.at[r]))

        plsc.subcore_barrier()

        def body(jg, carry):
            drain_scatter(2 * jg, rows0, sem0)
            gather(2 * jg + 2, rows0, sem0)
            drain_scatter(2 * jg + 1, rows1, sem1)
            gather(2 * jg + 3, rows1, sem1)
            return carry

        lax.fori_loop(0, (J - 3) // 2, body, 0)
        drain_scatter(J - 3, rows0, sem0)
        gather(J - 1, rows0, sem0)
        drain_scatter(J - 2, rows1, sem1)
        drain_scatter(J - 1, rows0, sem0)
        plsc.subcore_barrier()
        _tile_rows_copy(N, s, lambda r: pltpu.sync_copy(
            acc.at[r], out_hbm.at[c, r]))

    return scatter_kernel


def _make_sc_degree(N, E):
    EW = E // NW
    J = EW // KCH
    mesh = plsc.VectorSubcoreMesh(core_axis_name="c", subcore_axis_name="s")

    @functools.partial(
        pl.kernel,
        out_type=jax.ShapeDtypeStruct((NC, N, 8), jnp.float32),
        mesh=mesh,
        scratch_types=[
            pltpu.VMEM((J, KCH), jnp.int32),
            pltpu.VMEM((KCH, 8), jnp.float32),
            pltpu.VMEM_SHARED((N, 8), jnp.float32),
            pltpu.SemaphoreType.DMA,
        ],
    )
    def degree_kernel(init_hbm, onesk_hbm, dst_hbm, out_hbm,
                      dst_v, ones_v, acc, sem):
        c = lax.axis_index("c")
        s = lax.axis_index("s")
        w = c * NS + s

        _tile_rows_copy(N, s, lambda r: pltpu.sync_copy(
            init_hbm.at[c].at[r], acc.at[r]))

        pltpu.sync_copy(onesk_hbm, ones_v)
        pltpu.sync_copy(dst_hbm.at[w], dst_v)
        plsc.subcore_barrier()

        def body(j, carry):
            pltpu.async_copy(ones_v, acc.at[dst_v.at[j]], sem, add=True)
            return carry

        lax.fori_loop(0, J, body, 0)

        def drain(j, carry):
            pltpu.make_async_copy(ones_v, acc.at[dst_v.at[j]], sem).wait()
            return carry

        lax.fori_loop(0, J, drain, 0)
        plsc.subcore_barrier()
        _tile_rows_copy(N, s, lambda r: pltpu.sync_copy(
            acc.at[r], out_hbm.at[c, r]))

    return degree_kernel


_BR = 2000


def _row(H):
    return pl.BlockSpec((_BR, H), lambda i: (i, 0))


def _full(shape):
    return pl.BlockSpec(shape, lambda i: (0,) * len(shape))


def _acc2(H):
    return pl.BlockSpec((NC, _BR, H), lambda i: (0, i, 0))


def _tc_in_body(x_ref, win_ref, bin_ref, wc0_ref, dega_ref, degb_ref,
                h_ref, dinv_ref, xs_ref):
    h = jnp.dot(x_ref[...], win_ref[...],
                preferred_element_type=jnp.float32) + bin_ref[...]
    dinv = lax.rsqrt(dega_ref[...] + degb_ref[...])
    h_ref[...] = h
    dinv_ref[...] = dinv
    xs_ref[...] = dinv * jnp.dot(h, wc0_ref[...],
                                 preferred_element_type=jnp.float32)


def _tc_mid_body(has_xl, acc_ref, dinv_ref, bc_ref, h_ref, wl_ref, bl_ref,
                 *rest):
    if has_xl:
        xl_ref, wc_ref, hn_ref, xln_ref, xs_ref = rest
    else:
        wc_ref, hn_ref, xln_ref, xs_ref = rest
    dinv = dinv_ref[...]
    conv = dinv * (acc_ref[0] + acc_ref[1]) + bc_ref[...]
    lin = jnp.dot(h_ref[...], wl_ref[...],
                  preferred_element_type=jnp.float32) + bl_ref[...]
    hn = jnp.maximum(conv + lin, 0.0)
    xln = (xl_ref[...] + hn) if has_xl else hn
    hn_ref[...] = hn
    xln_ref[...] = xln
    xs_ref[...] = dinv * jnp.dot(hn, wc_ref[...],
                                 preferred_element_type=jnp.float32)


def _tc_out_body(acc_ref, dinv_ref, bc_ref, h_ref, wl_ref, bl_ref, xl_ref,
                 wp_ref, bp_ref, out_ref):
    conv = dinv_ref[...] * (acc_ref[0] + acc_ref[1]) + bc_ref[...]
    lin = jnp.dot(h_ref[...], wl_ref[...],
                  preferred_element_type=jnp.float32) + bl_ref[...]
    hn = jnp.maximum(conv + lin, 0.0)
    xl = xl_ref[...] + hn
    out_ref[...] = jnp.dot(xl, wp_ref[...],
                           preferred_element_type=jnp.float32) + bp_ref[...]


def kernel(x, edge_index, W_in, b_in, Wc, bc, Wl, bl, Wp, bp):
    N, _ = x.shape
    H = W_in.shape[1]
    E = edge_index.shape[1]
    DP = Wp.shape[1]
    grid = (N // _BR,)

    J = E // (NW * KCH)
    EP = J * NW * KCH
    src_i = edge_index[0].astype(jnp.int32)
    dst_i = edge_index[1].astype(jnp.int32)
    src = src_i.reshape(NW, E // NW)
    dst = dst_i.reshape(NW, J, KCH)
    zeros_nh = jnp.zeros((N, H), jnp.float32)
    init8 = jnp.stack([jnp.ones((N, 8), jnp.float32),
                       jnp.zeros((N, 8), jnp.float32)])
    ones_k8 = jnp.ones((KCH, 8), jnp.float32)

    sc_degree = _make_sc_degree(N, E)
    sc_scatter = _make_sc_scatter(N, EP, H)

    f32 = jnp.float32
    nh = jax.ShapeDtypeStruct((N, H), f32)

    deg2 = sc_degree(init8, ones_k8, dst)
    dega = deg2[0, :, 0:1]
    degb = deg2[1, :, 0:1]

    h0, dinv, xs = pl.pallas_call(
        _tc_in_body,
        grid=grid,
        in_specs=[_row(H), _full((H, H)), _full((1, H)), _full((H, H)),
                  _row(1), _row(1)],
        out_specs=[_row(H), _row(1), _row(H)],
        out_shape=[nh, jax.ShapeDtypeStruct((N, 1), f32), nh],
    )(x, W_in, b_in.reshape(1, H), Wc[0], dega, degb)

    h, xl = h0, None
    for i in range(2):
        acc = sc_scatter(xs, zeros_nh, src, dst)
        ins = [acc, dinv, bc[i].reshape(1, H), h, Wl[i], bl[i].reshape(1, H)]
        specs = [_acc2(H), _row(1), _full((1, H)), _row(H), _full((H, H)),
                 _full((1, H))]
        if xl is not None:
            ins.append(xl)
            specs.append(_row(H))
        ins.append(Wc[i + 1])
        specs.append(_full((H, H)))
        h, xl, xs = pl.pallas_call(
            functools.partial(_tc_mid_body, xl is not None),
            grid=grid,
            in_specs=specs,
            out_specs=[_row(H), _row(H), _row(H)],
            out_shape=[nh, nh, nh],
        )(*ins)

    acc = sc_scatter(xs, zeros_nh, src, dst)
    out = pl.pallas_call(
        _tc_out_body,
        grid=grid,
        in_specs=[_acc2(H), _row(1), _full((1, H)), _row(H), _full((H, H)),
                  _full((1, H)), _row(H), _full((H, DP)), _full((1, DP))],
        out_specs=_row(DP),
        out_shape=jax.ShapeDtypeStruct((N, DP), f32),
    )(acc, dinv, bc[2].reshape(1, H), h, Wl[2], bl[2].reshape(1, H), xl,
      Wp, bp.reshape(1, DP))
    return out
